# Initial kernel scaffold; baseline (speedup 1.0000x reference)
import jax
import jax.numpy as jnp
from jax.experimental import pallas as pl


def kernel(feat, edge_index, gamma, beta, W_ih, W_hh, b_ih, b_hh, W_self, W_neigh):
    raise NotImplementedError("write your pallas kernel here")


# speedup vs baseline: 2.5771x; 2.5771x over previous
import functools

import jax
import jax.numpy as jnp
from jax import lax
from jax.experimental import pallas as pl
from jax.experimental.pallas import tpu as pltpu
from jax.experimental.pallas import tpu_sc as plsc

N = 10000
T = 16
D = 256
G = 3 * D
E = N * T

BN = 1000
NB = N // BN

_EPS = 1e-5


def _stats_body(feat_ref, gamma_ref, beta_ref, scale_ref, shift_ref):
    f = feat_ref[...]
    mean = jnp.mean(f, axis=0, keepdims=True)
    var = jnp.mean(f * f, axis=0, keepdims=True) - mean * mean
    scale = gamma_ref[...] * lax.rsqrt(var + _EPS)
    shift = beta_ref[...] - mean * scale
    scale_ref[...] = jnp.broadcast_to(scale, scale_ref.shape)
    shift_ref[...] = jnp.broadcast_to(shift, shift_ref.shape)


def _stats(feat, gamma2, beta2):
    return pl.pallas_call(
        _stats_body,
        out_shape=(
            jax.ShapeDtypeStruct((8, D), jnp.float32),
            jax.ShapeDtypeStruct((8, D), jnp.float32),
        ),
    )(feat, gamma2, beta2)


def _proj_body(feat_ref, w_ih_ref, b_ih_ref, scale_ref, shift_ref, p_ref):
    s = scale_ref[0:1, :]
    sh = shift_ref[0:1, :]
    w = w_ih_ref[...] * s
    bias = b_ih_ref[...] + lax.dot_general(
        sh, w_ih_ref[...], (((1,), (1,)), ((), ())),
        preferred_element_type=jnp.float32)
    p = lax.dot_general(
        feat_ref[...], w, (((1,), (1,)), ((), ())),
        preferred_element_type=jnp.float32)
    p_ref[...] = p + bias


def _proj(feat, w_ih, b_ih2, scale, shift):
    return pl.pallas_call(
        _proj_body,
        grid=(NB,),
        in_specs=[
            pl.BlockSpec((BN, D), lambda i: (i, 0)),
            pl.BlockSpec((G, D), lambda i: (0, 0)),
            pl.BlockSpec((1, G), lambda i: (0, 0)),
            pl.BlockSpec((8, D), lambda i: (0, 0)),
            pl.BlockSpec((8, D), lambda i: (0, 0)),
        ],
        out_specs=pl.BlockSpec((BN, G), lambda i: (i, 0)),
        out_shape=jax.ShapeDtypeStruct((N, G), jnp.float32),
        compiler_params=pltpu.CompilerParams(
            dimension_semantics=("parallel",)),
    )(feat, w_ih, b_ih2, scale, shift)


_NC, _NS = 2, 16
NW = _NC * _NS
BPW = E // NW
CH = 40
NCHUNK = BPW // CH


def _gather_body(p_hbm, idx_hbm, out_hbm, idx_v, buf0, buf1, sem0, sem1):
    wid = lax.axis_index("s") * _NC + lax.axis_index("c")
    base = wid * BPW
    pltpu.sync_copy(idx_hbm.at[pl.ds(base, BPW)], idx_v)

    pltpu.async_copy(p_hbm.at[idx_v.at[pl.ds(0, CH)]], buf0, sem0)
    pltpu.async_copy(p_hbm.at[idx_v.at[pl.ds(CH, CH)]], buf1, sem1)

    def outer(jj, _):
        for b, (buf, sem) in enumerate(((buf0, sem0), (buf1, sem1))):
            j = jj * 2 + b
            pltpu.make_async_copy(
                p_hbm.at[idx_v.at[pl.ds(j * CH, CH)]], buf, sem).wait()
            pltpu.sync_copy(buf, out_hbm.at[pl.ds(base + j * CH, CH)])

            @pl.when(j + 2 < NCHUNK)
            def _():
                pltpu.async_copy(
                    p_hbm.at[idx_v.at[pl.ds((j + 2) * CH, CH)]], buf, sem)
        return 0

    lax.fori_loop(0, (NCHUNK - 1) // 2, outer, 0)
    j = NCHUNK - 1
    pltpu.make_async_copy(
        p_hbm.at[idx_v.at[pl.ds(j * CH, CH)]], buf0, sem0).wait()
    pltpu.sync_copy(buf0, out_hbm.at[pl.ds(base + j * CH, CH)])


@functools.cache
def _gather():
    return pl.kernel(
        _gather_body,
        mesh=plsc.VectorSubcoreMesh(core_axis_name="c", subcore_axis_name="s",
                                    num_cores=_NC, num_subcores=_NS),
        out_type=jax.ShapeDtypeStruct((E, G), jnp.float32),
        scratch_types=[
            pltpu.VMEM((BPW,), jnp.int32),
            pltpu.VMEM((CH, G), jnp.float32),
            pltpu.VMEM((CH, G), jnp.float32),
            pltpu.SemaphoreType.DMA,
            pltpu.SemaphoreType.DMA,
        ],
    )


def _gru_body(x_ref, feat_ref, whh_ref, bhh_ref, scale_ref, shift_ref,
              wself_ref, wneigh_ref, out_ref, h_ref):
    t = pl.program_id(1)

    @pl.when(t == 0)
    def _():
        h_ref[...] = jnp.zeros_like(h_ref)

    h = h_ref[...]
    x = x_ref[0]
    gh = lax.dot_general(
        h, whh_ref[...], (((1,), (1,)), ((), ())),
        preferred_element_type=jnp.float32) + bhh_ref[...]
    r = jax.nn.sigmoid(x[:, :D] + gh[:, :D])
    z = jax.nn.sigmoid(x[:, D:2 * D] + gh[:, D:2 * D])
    n = jnp.tanh(x[:, 2 * D:] + r * gh[:, 2 * D:])
    h_new = (1.0 - z) * n + z * h
    h_ref[...] = h_new

    @pl.when(t == T - 1)
    def _():
        s = scale_ref[0:1, :]
        sh = shift_ref[0:1, :]
        ws = wself_ref[...] * s
        bias = lax.dot_general(
            sh, wself_ref[...], (((1,), (1,)), ((), ())),
            preferred_element_type=jnp.float32)
        out_ref[...] = (
            lax.dot_general(feat_ref[...], ws, (((1,), (1,)), ((), ())),
                            preferred_element_type=jnp.float32)
            + bias
            + lax.dot_general(h_new, wneigh_ref[...], (((1,), (1,)), ((), ())),
                              preferred_element_type=jnp.float32))


def _gru(x, feat, w_hh, b_hh2, scale, shift, w_self, w_neigh):
    return pl.pallas_call(
        _gru_body,
        grid=(NB, T),
        in_specs=[
            pl.BlockSpec((1, BN, G), lambda i, t: (t, i, 0)),
            pl.BlockSpec((BN, D), lambda i, t: (i, 0)),
            pl.BlockSpec((G, D), lambda i, t: (0, 0)),
            pl.BlockSpec((1, G), lambda i, t: (0, 0)),
            pl.BlockSpec((8, D), lambda i, t: (0, 0)),
            pl.BlockSpec((8, D), lambda i, t: (0, 0)),
            pl.BlockSpec((D, D), lambda i, t: (0, 0)),
            pl.BlockSpec((D, D), lambda i, t: (0, 0)),
        ],
        out_specs=pl.BlockSpec((BN, D), lambda i, t: (i, 0)),
        out_shape=jax.ShapeDtypeStruct((N, D), jnp.float32),
        scratch_shapes=[pltpu.VMEM((BN, D), jnp.float32)],
        compiler_params=pltpu.CompilerParams(
            dimension_semantics=("parallel", "arbitrary")),
    )(x, feat, w_hh, b_hh2, scale, shift, w_self, w_neigh)


def kernel(feat, edge_index, gamma, beta, W_ih, W_hh, b_ih, b_hh,
           W_self, W_neigh):
    src = edge_index[0].astype(jnp.int32)
    src_t = src.reshape(N, T).T.reshape(E)

    scale, shift = _stats(feat, gamma.reshape(1, D), beta.reshape(1, D))
    p = _proj(feat, W_ih, b_ih.reshape(1, G), scale, shift)
    x = _gather()(p, src_t).reshape(T, N, G)
    return _gru(x, feat, W_hh, b_hh.reshape(1, G), scale, shift,
                W_self, W_neigh)


# speedup vs baseline: 4.4238x; 1.7166x over previous
import functools

import jax
import jax.numpy as jnp
from jax import lax
from jax.experimental import pallas as pl
from jax.experimental.pallas import tpu as pltpu
from jax.experimental.pallas import tpu_sc as plsc

N = 10000
T = 16
D = 256
G = 3 * D
E = N * T

BN = 1000
NB = N // BN

_EPS = 1e-5


def _prep_body(feat_ref, gamma_ref, beta_ref, w_ih_ref, b_ih_ref,
               w_self_ref, feat_bf_ref, w_ihs_ref, bias_ih_ref,
               w_selfs_ref, bias_self_ref):
    f = feat_ref[...]
    mean = jnp.mean(f, axis=0, keepdims=True)
    var = jnp.mean(f * f, axis=0, keepdims=True) - mean * mean
    scale = gamma_ref[...] * lax.rsqrt(var + _EPS)
    shift = beta_ref[...] - mean * scale
    feat_bf_ref[...] = f.astype(jnp.bfloat16)
    w_ih = w_ih_ref[...]
    w_ihs_ref[...] = (w_ih * scale).astype(jnp.bfloat16)
    bias_ih_ref[...] = b_ih_ref[...] + lax.dot_general(
        shift, w_ih, (((1,), (1,)), ((), ())),
        preferred_element_type=jnp.float32)
    w_self = w_self_ref[...]
    w_selfs_ref[...] = (w_self * scale).astype(jnp.bfloat16)
    bias_self_ref[...] = lax.dot_general(
        shift, w_self, (((1,), (1,)), ((), ())),
        preferred_element_type=jnp.float32)


def _prep(feat, gamma2, beta2, w_ih, b_ih2, w_self):
    return pl.pallas_call(
        _prep_body,
        out_shape=(
            jax.ShapeDtypeStruct((N, D), jnp.bfloat16),
            jax.ShapeDtypeStruct((G, D), jnp.bfloat16),
            jax.ShapeDtypeStruct((1, G), jnp.float32),
            jax.ShapeDtypeStruct((D, D), jnp.bfloat16),
            jax.ShapeDtypeStruct((1, D), jnp.float32),
        ),
    )(feat, gamma2, beta2, w_ih, b_ih2, w_self)


_NC, _NS = 2, 16
NW = _NC * _NS
BPW = E // NW
CH = 200
NCHUNK = BPW // CH


def _gather_body(src_hbm, idx_hbm, out_hbm, idx_v, buf0, buf1,
                 gsem0, gsem1):
    wid = lax.axis_index("s") * _NC + lax.axis_index("c")
    base = wid * BPW
    pltpu.sync_copy(idx_hbm.at[pl.ds(base, BPW)], idx_v)

    pltpu.async_copy(src_hbm.at[idx_v.at[pl.ds(0, CH)]], buf0, gsem0)
    pltpu.async_copy(src_hbm.at[idx_v.at[pl.ds(CH, CH)]], buf1, gsem1)

    def outer(jj, _):
        for b, (buf, gsem) in enumerate(((buf0, gsem0), (buf1, gsem1))):
            j = jj * 2 + b
            pltpu.make_async_copy(
                src_hbm.at[idx_v.at[pl.ds(j * CH, CH)]], buf, gsem).wait()
            pltpu.sync_copy(buf, out_hbm.at[pl.ds(base + j * CH, CH)])

            @pl.when(j + 2 < NCHUNK)
            def _():
                pltpu.async_copy(
                    src_hbm.at[idx_v.at[pl.ds((j + 2) * CH, CH)]], buf, gsem)
        return 0

    lax.fori_loop(0, (NCHUNK - 1) // 2, outer, 0)
    j = NCHUNK - 1
    pltpu.make_async_copy(
        src_hbm.at[idx_v.at[pl.ds(j * CH, CH)]], buf0, gsem0).wait()
    pltpu.sync_copy(buf0, out_hbm.at[pl.ds(base + j * CH, CH)])


@functools.cache
def _gather():
    return pl.kernel(
        _gather_body,
        mesh=plsc.VectorSubcoreMesh(core_axis_name="c", subcore_axis_name="s",
                                    num_cores=_NC, num_subcores=_NS),
        out_type=jax.ShapeDtypeStruct((E, D), jnp.float32),
        scratch_types=[
            pltpu.VMEM((BPW,), jnp.int32),
            pltpu.VMEM((CH, D), jnp.float32),
            pltpu.VMEM((CH, D), jnp.float32),
            pltpu.SemaphoreType.DMA,
            pltpu.SemaphoreType.DMA,
        ],
    )


def _gru_body(mail_ref, feat_bf_ref, w_ihs_ref, bias_ih_ref, whh_ref,
              bhh_ref, w_selfs_ref, bias_self_ref, wneigh_ref,
              out_ref, h_ref):
    t = pl.program_id(1)

    @pl.when(t == 0)
    def _():
        h_ref[...] = jnp.zeros_like(h_ref)

    h = h_ref[...]
    x = lax.dot_general(
        mail_ref[0].astype(jnp.bfloat16), w_ihs_ref[...],
        (((1,), (1,)), ((), ())),
        preferred_element_type=jnp.float32) + bias_ih_ref[...]
    gh = lax.dot_general(
        h.astype(jnp.bfloat16), whh_ref[...], (((1,), (1,)), ((), ())),
        preferred_element_type=jnp.float32) + bhh_ref[...]
    r = jax.nn.sigmoid(x[:, :D] + gh[:, :D])
    z = jax.nn.sigmoid(x[:, D:2 * D] + gh[:, D:2 * D])
    n = jnp.tanh(x[:, 2 * D:] + r * gh[:, 2 * D:])
    h_new = (1.0 - z) * n + z * h
    h_ref[...] = h_new

    @pl.when(t == T - 1)
    def _():
        out_ref[...] = (
            lax.dot_general(feat_bf_ref[...], w_selfs_ref[...],
                            (((1,), (1,)), ((), ())),
                            preferred_element_type=jnp.float32)
            + bias_self_ref[...]
            + lax.dot_general(h_new.astype(jnp.bfloat16), wneigh_ref[...],
                              (((1,), (1,)), ((), ())),
                              preferred_element_type=jnp.float32))


def _gru(mail, feat_bf, w_ihs, bias_ih, w_hh_bf, b_hh2, w_selfs, bias_self,
         w_neigh_bf):
    return pl.pallas_call(
        _gru_body,
        grid=(NB, T),
        in_specs=[
            pl.BlockSpec((1, BN, D), lambda i, t: (t, i, 0)),
            pl.BlockSpec((BN, D), lambda i, t: (i, 0)),
            pl.BlockSpec((G, D), lambda i, t: (0, 0)),
            pl.BlockSpec((1, G), lambda i, t: (0, 0)),
            pl.BlockSpec((G, D), lambda i, t: (0, 0)),
            pl.BlockSpec((1, G), lambda i, t: (0, 0)),
            pl.BlockSpec((D, D), lambda i, t: (0, 0)),
            pl.BlockSpec((1, D), lambda i, t: (0, 0)),
            pl.BlockSpec((D, D), lambda i, t: (0, 0)),
        ],
        out_specs=pl.BlockSpec((BN, D), lambda i, t: (i, 0)),
        out_shape=jax.ShapeDtypeStruct((N, D), jnp.float32),
        scratch_shapes=[pltpu.VMEM((BN, D), jnp.float32)],
        compiler_params=pltpu.CompilerParams(
            dimension_semantics=("parallel", "arbitrary")),
    )(mail, feat_bf, w_ihs, bias_ih, w_hh_bf, b_hh2, w_selfs, bias_self,
      w_neigh_bf)


def kernel(feat, edge_index, gamma, beta, W_ih, W_hh, b_ih, b_hh,
           W_self, W_neigh):
    src = edge_index[0].astype(jnp.int32)
    src_t = src.reshape(N, T).T.reshape(E)

    feat_bf, w_ihs, bias_ih, w_selfs, bias_self = _prep(
        feat, gamma.reshape(1, D), beta.reshape(1, D), W_ih,
        b_ih.reshape(1, G), W_self)
    mail = _gather()(feat, src_t).reshape(T, N, D)
    return _gru(mail, feat_bf, w_ihs, bias_ih,
                W_hh.astype(jnp.bfloat16), b_hh.reshape(1, G),
                w_selfs, bias_self, W_neigh.astype(jnp.bfloat16))


# speedup vs baseline: 5.1448x; 1.1630x over previous
import functools

import jax
import jax.numpy as jnp
from jax import lax
from jax.experimental import pallas as pl
from jax.experimental.pallas import tpu as pltpu
from jax.experimental.pallas import tpu_sc as plsc

N = 10000
T = 16
D = 256
G = 3 * D
E = N * T

BN = 2000
NB = N // BN

_EPS = 1e-5


def _prep_body(feat_ref, gamma_ref, beta_ref, w_ih_ref, b_ih_ref,
               w_self_ref, feat_bf_ref, w_ihs_ref, bias_ih_ref,
               w_selfs_ref, bias_self_ref):
    f = feat_ref[...]
    mean = jnp.mean(f, axis=0, keepdims=True)
    var = jnp.mean(f * f, axis=0, keepdims=True) - mean * mean
    scale = gamma_ref[...] * lax.rsqrt(var + _EPS)
    shift = beta_ref[...] - mean * scale
    feat_bf_ref[...] = f.astype(jnp.bfloat16)
    w_ih = w_ih_ref[...]
    w_ihs_ref[...] = (w_ih * scale).astype(jnp.bfloat16)
    bias_ih_ref[...] = b_ih_ref[...] + lax.dot_general(
        shift, w_ih, (((1,), (1,)), ((), ())),
        preferred_element_type=jnp.float32)
    w_self = w_self_ref[...]
    w_selfs_ref[...] = (w_self * scale).astype(jnp.bfloat16)
    bias_self_ref[...] = lax.dot_general(
        shift, w_self, (((1,), (1,)), ((), ())),
        preferred_element_type=jnp.float32)


def _prep(feat, gamma2, beta2, w_ih, b_ih2, w_self):
    return pl.pallas_call(
        _prep_body,
        out_shape=(
            jax.ShapeDtypeStruct((N, D), jnp.bfloat16),
            jax.ShapeDtypeStruct((G, D), jnp.bfloat16),
            jax.ShapeDtypeStruct((1, G), jnp.float32),
            jax.ShapeDtypeStruct((D, D), jnp.bfloat16),
            jax.ShapeDtypeStruct((1, D), jnp.float32),
        ),
    )(feat, gamma2, beta2, w_ih, b_ih2, w_self)


_NC, _NS = 2, 16
NW = _NC * _NS
TSPLIT = 2
EH = E // TSPLIT
CH = 128
NCHUNK = EH // CH
NK = -(-NCHUNK // NW)


def _gather_body(src_hbm, idx_hbm, out_hbm, idx_v, buf0, buf1,
                 gsem0, gsem1):
    wid = lax.axis_index("s") * _NC + lax.axis_index("c")
    nk_me = jnp.where(wid < NCHUNK - NW * (NK - 1), NK, NK - 1)
    pltpu.sync_copy(idx_hbm.at[wid], idx_v)

    bufs = ((buf0, gsem0), (buf1, gsem1))
    pltpu.async_copy(src_hbm.at[idx_v.at[0]], buf0, gsem0)
    pltpu.async_copy(src_hbm.at[idx_v.at[1]], buf1, gsem1)
    for k in range(NK):
        buf, gsem = bufs[k % 2]
        row = (wid + NW * k) * CH

        @pl.when(k < nk_me)
        def _():
            pltpu.make_async_copy(
                src_hbm.at[idx_v.at[k]], buf, gsem).wait()
            pltpu.sync_copy(buf, out_hbm.at[pl.ds(row, CH)])

        if k + 2 < NK:
            @pl.when(k + 2 < nk_me)
            def _():
                pltpu.async_copy(src_hbm.at[idx_v.at[k + 2]], buf, gsem)


@functools.cache
def _gather():
    return pl.kernel(
        _gather_body,
        mesh=plsc.VectorSubcoreMesh(core_axis_name="c", subcore_axis_name="s",
                                    num_cores=_NC, num_subcores=_NS),
        out_type=jax.ShapeDtypeStruct((EH, D), jnp.float32),
        scratch_types=[
            pltpu.VMEM((NK, CH), jnp.int32),
            pltpu.VMEM((CH, D), jnp.float32),
            pltpu.VMEM((CH, D), jnp.float32),
            pltpu.SemaphoreType.DMA,
            pltpu.SemaphoreType.DMA,
        ],
    )


TH = T // TSPLIT


def _gru_step(mail_ref, h, w_ihs_ref, bias_ih_ref, whh_ref, bhh_ref):
    x = lax.dot_general(
        mail_ref[0].astype(jnp.bfloat16), w_ihs_ref[...],
        (((1,), (1,)), ((), ())),
        preferred_element_type=jnp.float32) + bias_ih_ref[...]
    gh = lax.dot_general(
        h.astype(jnp.bfloat16), whh_ref[...], (((1,), (1,)), ((), ())),
        preferred_element_type=jnp.float32) + bhh_ref[...]
    rz = x[:, :2 * D] + gh[:, :2 * D]
    srz = 0.5 * jnp.tanh(0.5 * rz) + 0.5
    r = srz[:, :D]
    z = srz[:, D:]
    n = jnp.tanh(x[:, 2 * D:] + r * gh[:, 2 * D:])
    return z * (h - n) + n


def _gru1_body(mail_ref, w_ihs_ref, bias_ih_ref, whh_ref, bhh_ref,
               hout_ref, h_ref):
    t = pl.program_id(1)

    @pl.when(t == 0)
    def _():
        h_ref[...] = jnp.zeros_like(h_ref)

    h_new = _gru_step(mail_ref, h_ref[...], w_ihs_ref, bias_ih_ref,
                      whh_ref, bhh_ref)
    h_ref[...] = h_new

    @pl.when(t == TH - 1)
    def _():
        hout_ref[...] = h_new


def _gru2_body(mail_ref, hin_ref, feat_bf_ref, w_ihs_ref, bias_ih_ref,
               whh_ref, bhh_ref, w_selfs_ref, bias_self_ref, wneigh_ref,
               out_ref, h_ref):
    t = pl.program_id(1)

    @pl.when(t == 0)
    def _():
        h_ref[...] = hin_ref[...]

    h_new = _gru_step(mail_ref, h_ref[...], w_ihs_ref, bias_ih_ref,
                      whh_ref, bhh_ref)
    h_ref[...] = h_new

    @pl.when(t == TH - 1)
    def _():
        out_ref[...] = (
            lax.dot_general(feat_bf_ref[...], w_selfs_ref[...],
                            (((1,), (1,)), ((), ())),
                            preferred_element_type=jnp.float32)
            + bias_self_ref[...]
            + lax.dot_general(h_new.astype(jnp.bfloat16), wneigh_ref[...],
                              (((1,), (1,)), ((), ())),
                              preferred_element_type=jnp.float32))


_SMALL_SPECS = [
    pl.BlockSpec((G, D), lambda i, t: (0, 0)),
    pl.BlockSpec((1, G), lambda i, t: (0, 0)),
    pl.BlockSpec((G, D), lambda i, t: (0, 0)),
    pl.BlockSpec((1, G), lambda i, t: (0, 0)),
]


def _gru1(mailA, w_ihs, bias_ih, w_hh_bf, b_hh2):
    return pl.pallas_call(
        _gru1_body,
        grid=(NB, TH),
        in_specs=[pl.BlockSpec((1, BN, D), lambda i, t: (t, i, 0))]
        + _SMALL_SPECS,
        out_specs=pl.BlockSpec((BN, D), lambda i, t: (i, 0)),
        out_shape=jax.ShapeDtypeStruct((N, D), jnp.float32),
        scratch_shapes=[pltpu.VMEM((BN, D), jnp.float32)],
        compiler_params=pltpu.CompilerParams(
            dimension_semantics=("parallel", "arbitrary")),
    )(mailA, w_ihs, bias_ih, w_hh_bf, b_hh2)


def _gru2(mailB, h_mid, feat_bf, w_ihs, bias_ih, w_hh_bf, b_hh2,
          w_selfs, bias_self, w_neigh_bf):
    return pl.pallas_call(
        _gru2_body,
        grid=(NB, TH),
        in_specs=[
            pl.BlockSpec((1, BN, D), lambda i, t: (t, i, 0)),
            pl.BlockSpec((BN, D), lambda i, t: (i, 0)),
            pl.BlockSpec((BN, D), lambda i, t: (i, 0)),
        ] + _SMALL_SPECS + [
            pl.BlockSpec((D, D), lambda i, t: (0, 0)),
            pl.BlockSpec((1, D), lambda i, t: (0, 0)),
            pl.BlockSpec((D, D), lambda i, t: (0, 0)),
        ],
        out_specs=pl.BlockSpec((BN, D), lambda i, t: (i, 0)),
        out_shape=jax.ShapeDtypeStruct((N, D), jnp.float32),
        scratch_shapes=[pltpu.VMEM((BN, D), jnp.float32)],
        compiler_params=pltpu.CompilerParams(
            dimension_semantics=("parallel", "arbitrary")),
    )(mailB, h_mid, feat_bf, w_ihs, bias_ih, w_hh_bf, b_hh2,
      w_selfs, bias_self, w_neigh_bf)


def kernel(feat, edge_index, gamma, beta, W_ih, W_hh, b_ih, b_hh,
           W_self, W_neigh):
    src = edge_index[0].astype(jnp.int32)
    src_t = src.reshape(N, T).T.reshape(E)
    cid = jnp.minimum(jnp.arange(NW)[:, None] + NW * jnp.arange(NK)[None, :],
                      NCHUNK - 1)
    src_a = src_t[:EH].reshape(NCHUNK, CH)[cid]
    src_b = src_t[EH:].reshape(NCHUNK, CH)[cid]

    mail_a = _gather()(feat, src_a).reshape(TH, N, D)
    mail_b = _gather()(feat, src_b).reshape(TH, N, D)
    feat_bf, w_ihs, bias_ih, w_selfs, bias_self = _prep(
        feat, gamma.reshape(1, D), beta.reshape(1, D), W_ih,
        b_ih.reshape(1, G), W_self)
    w_hh_bf = W_hh.astype(jnp.bfloat16)
    b_hh2 = b_hh.reshape(1, G)
    h_mid = _gru1(mail_a, w_ihs, bias_ih, w_hh_bf, b_hh2)
    return _gru2(mail_b, h_mid, feat_bf, w_ihs, bias_ih, w_hh_bf, b_hh2,
                 w_selfs, bias_self, W_neigh.astype(jnp.bfloat16))


# speedup vs baseline: 5.7599x; 1.1195x over previous
import functools

import jax
import jax.numpy as jnp
from jax import lax
from jax.experimental import pallas as pl
from jax.experimental.pallas import tpu as pltpu
from jax.experimental.pallas import tpu_sc as plsc

N = 10000
T = 16
D = 256
G = 3 * D
E = N * T

BN = 2000
NB = N // BN

_EPS = 1e-5


def _prep_body(feat_ref, gamma_ref, beta_ref, w_ih_ref, b_ih_ref,
               w_self_ref, feat_bf_ref, feat_pk_ref, w_ihs_ref, bias_ih_ref,
               w_selfs_ref, bias_self_ref):
    f = feat_ref[...]
    mean = jnp.mean(f, axis=0, keepdims=True)
    var = jnp.mean(f * f, axis=0, keepdims=True) - mean * mean
    scale = gamma_ref[...] * lax.rsqrt(var + _EPS)
    shift = beta_ref[...] - mean * scale
    feat_bf_ref[...] = f.astype(jnp.bfloat16)
    u = lax.bitcast_convert_type(f, jnp.int32)
    ubf = jnp.right_shift(u + 0x7FFF + (jnp.right_shift(u, 16) & 1), 16)
    feat_pk_ref[...] = pltpu.pack_elementwise(
        [ubf[:, :D // 2], ubf[:, D // 2:]], packed_dtype=jnp.int16)
    w_ih = w_ih_ref[...]
    w_ihs_ref[...] = (w_ih * scale).astype(jnp.bfloat16)
    bias_ih_ref[...] = b_ih_ref[...] + lax.dot_general(
        shift, w_ih, (((1,), (1,)), ((), ())),
        preferred_element_type=jnp.float32)
    w_self = w_self_ref[...]
    w_selfs_ref[...] = (w_self * scale).astype(jnp.bfloat16)
    bias_self_ref[...] = lax.dot_general(
        shift, w_self, (((1,), (1,)), ((), ())),
        preferred_element_type=jnp.float32)


def _prep(feat, gamma2, beta2, w_ih, b_ih2, w_self):
    return pl.pallas_call(
        _prep_body,
        out_shape=(
            jax.ShapeDtypeStruct((N, D), jnp.bfloat16),
            jax.ShapeDtypeStruct((N, D // 2), jnp.int32),
            jax.ShapeDtypeStruct((G, D), jnp.bfloat16),
            jax.ShapeDtypeStruct((1, G), jnp.float32),
            jax.ShapeDtypeStruct((D, D), jnp.bfloat16),
            jax.ShapeDtypeStruct((1, D), jnp.float32),
        ),
    )(feat, gamma2, beta2, w_ih, b_ih2, w_self)


_NC, _NS = 2, 16
NW = _NC * _NS
TSPLIT = 2
EH = E // TSPLIT
CH = 128
NCHUNK = EH // CH
NK = -(-NCHUNK // NW)


def _gather_body(src_hbm, idx_hbm, out_hbm, idx_v, buf0, buf1,
                 gsem0, gsem1):
    wid = lax.axis_index("s") * _NC + lax.axis_index("c")
    nk_me = jnp.where(wid < NCHUNK - NW * (NK - 1), NK, NK - 1)
    pltpu.sync_copy(idx_hbm.at[wid], idx_v)

    bufs = ((buf0, gsem0), (buf1, gsem1))
    pltpu.async_copy(src_hbm.at[idx_v.at[0]], buf0, gsem0)
    pltpu.async_copy(src_hbm.at[idx_v.at[1]], buf1, gsem1)
    for k in range(NK):
        buf, gsem = bufs[k % 2]
        row = (wid + NW * k) * CH

        @pl.when(k < nk_me)
        def _():
            pltpu.make_async_copy(
                src_hbm.at[idx_v.at[k]], buf, gsem).wait()
            pltpu.sync_copy(buf, out_hbm.at[pl.ds(row, CH)])

        if k + 2 < NK:
            @pl.when(k + 2 < nk_me)
            def _():
                pltpu.async_copy(src_hbm.at[idx_v.at[k + 2]], buf, gsem)


@functools.cache
def _gather():
    return pl.kernel(
        _gather_body,
        mesh=plsc.VectorSubcoreMesh(core_axis_name="c", subcore_axis_name="s",
                                    num_cores=_NC, num_subcores=_NS),
        out_type=jax.ShapeDtypeStruct((EH, D // 2), jnp.int32),
        scratch_types=[
            pltpu.VMEM((NK, CH), jnp.int32),
            pltpu.VMEM((CH, D // 2), jnp.int32),
            pltpu.VMEM((CH, D // 2), jnp.int32),
            pltpu.SemaphoreType.DMA,
            pltpu.SemaphoreType.DMA,
        ],
    )


TH = T // TSPLIT


def _gru_step(mail_ref, h, w_ihs_ref, bias_ih_ref, whh_ref, bhh_ref):
    pk = mail_ref[0]
    lo = pltpu.unpack_elementwise(
        pk, index=0, packed_dtype=jnp.int16, unpacked_dtype=jnp.int32)
    hi = pltpu.unpack_elementwise(
        pk, index=1, packed_dtype=jnp.int16, unpacked_dtype=jnp.int32)
    mail_bf = jnp.concatenate(
        [lax.bitcast_convert_type(lax.shift_left(lo, 16), jnp.float32),
         lax.bitcast_convert_type(lax.shift_left(hi, 16), jnp.float32)],
        axis=1).astype(jnp.bfloat16)
    x = lax.dot_general(
        mail_bf, w_ihs_ref[...],
        (((1,), (1,)), ((), ())),
        preferred_element_type=jnp.float32) + bias_ih_ref[...]
    gh = lax.dot_general(
        h.astype(jnp.bfloat16), whh_ref[...], (((1,), (1,)), ((), ())),
        preferred_element_type=jnp.float32) + bhh_ref[...]
    rz = x[:, :2 * D] + gh[:, :2 * D]
    srz = 0.5 * jnp.tanh(0.5 * rz) + 0.5
    r = srz[:, :D]
    z = srz[:, D:]
    n = jnp.tanh(x[:, 2 * D:] + r * gh[:, 2 * D:])
    return z * (h - n) + n


def _gru1_body(mail_ref, w_ihs_ref, bias_ih_ref, whh_ref, bhh_ref,
               hout_ref, h_ref):
    t = pl.program_id(1)

    @pl.when(t == 0)
    def _():
        h_ref[...] = jnp.zeros_like(h_ref)

    h_new = _gru_step(mail_ref, h_ref[...], w_ihs_ref, bias_ih_ref,
                      whh_ref, bhh_ref)
    h_ref[...] = h_new

    @pl.when(t == TH - 1)
    def _():
        hout_ref[...] = h_new


def _gru2_body(mail_ref, hin_ref, feat_bf_ref, w_ihs_ref, bias_ih_ref,
               whh_ref, bhh_ref, w_selfs_ref, bias_self_ref, wneigh_ref,
               out_ref, h_ref):
    t = pl.program_id(1)

    @pl.when(t == 0)
    def _():
        h_ref[...] = hin_ref[...]

    h_new = _gru_step(mail_ref, h_ref[...], w_ihs_ref, bias_ih_ref,
                      whh_ref, bhh_ref)
    h_ref[...] = h_new

    @pl.when(t == TH - 1)
    def _():
        out_ref[...] = (
            lax.dot_general(feat_bf_ref[...], w_selfs_ref[...],
                            (((1,), (1,)), ((), ())),
                            preferred_element_type=jnp.float32)
            + bias_self_ref[...]
            + lax.dot_general(h_new.astype(jnp.bfloat16), wneigh_ref[...],
                              (((1,), (1,)), ((), ())),
                              preferred_element_type=jnp.float32))


_SMALL_SPECS = [
    pl.BlockSpec((G, D), lambda i, t: (0, 0)),
    pl.BlockSpec((1, G), lambda i, t: (0, 0)),
    pl.BlockSpec((G, D), lambda i, t: (0, 0)),
    pl.BlockSpec((1, G), lambda i, t: (0, 0)),
]


def _gru1(mailA, w_ihs, bias_ih, w_hh_bf, b_hh2):
    return pl.pallas_call(
        _gru1_body,
        grid=(NB, TH),
        in_specs=[pl.BlockSpec((1, BN, D // 2), lambda i, t: (t, i, 0))]
        + _SMALL_SPECS,
        out_specs=pl.BlockSpec((BN, D), lambda i, t: (i, 0)),
        out_shape=jax.ShapeDtypeStruct((N, D), jnp.float32),
        scratch_shapes=[pltpu.VMEM((BN, D), jnp.float32)],
        compiler_params=pltpu.CompilerParams(
            dimension_semantics=("parallel", "arbitrary")),
    )(mailA, w_ihs, bias_ih, w_hh_bf, b_hh2)


def _gru2(mailB, h_mid, feat_bf, w_ihs, bias_ih, w_hh_bf, b_hh2,
          w_selfs, bias_self, w_neigh_bf):
    return pl.pallas_call(
        _gru2_body,
        grid=(NB, TH),
        in_specs=[
            pl.BlockSpec((1, BN, D // 2), lambda i, t: (t, i, 0)),
            pl.BlockSpec((BN, D), lambda i, t: (i, 0)),
            pl.BlockSpec((BN, D), lambda i, t: (i, 0)),
        ] + _SMALL_SPECS + [
            pl.BlockSpec((D, D), lambda i, t: (0, 0)),
            pl.BlockSpec((1, D), lambda i, t: (0, 0)),
            pl.BlockSpec((D, D), lambda i, t: (0, 0)),
        ],
        out_specs=pl.BlockSpec((BN, D), lambda i, t: (i, 0)),
        out_shape=jax.ShapeDtypeStruct((N, D), jnp.float32),
        scratch_shapes=[pltpu.VMEM((BN, D), jnp.float32)],
        compiler_params=pltpu.CompilerParams(
            dimension_semantics=("parallel", "arbitrary")),
    )(mailB, h_mid, feat_bf, w_ihs, bias_ih, w_hh_bf, b_hh2,
      w_selfs, bias_self, w_neigh_bf)


def kernel(feat, edge_index, gamma, beta, W_ih, W_hh, b_ih, b_hh,
           W_self, W_neigh):
    src = edge_index[0].astype(jnp.int32)
    src_t = src.reshape(N, T).T.reshape(E)
    cid = jnp.minimum(jnp.arange(NW)[:, None] + NW * jnp.arange(NK)[None, :],
                      NCHUNK - 1)
    src_a = src_t[:EH].reshape(NCHUNK, CH)[cid]
    src_b = src_t[EH:].reshape(NCHUNK, CH)[cid]

    feat_bf, feat_pk, w_ihs, bias_ih, w_selfs, bias_self = _prep(
        feat, gamma.reshape(1, D), beta.reshape(1, D), W_ih,
        b_ih.reshape(1, G), W_self)
    mail_a = _gather()(feat_pk, src_a).reshape(TH, N, D // 2)
    mail_b = _gather()(feat_pk, src_b).reshape(TH, N, D // 2)
    w_hh_bf = W_hh.astype(jnp.bfloat16)
    b_hh2 = b_hh.reshape(1, G)
    h_mid = _gru1(mail_a, w_ihs, bias_ih, w_hh_bf, b_hh2)
    return _gru2(mail_b, h_mid, feat_bf, w_ihs, bias_ih, w_hh_bf, b_hh2,
                 w_selfs, bias_self, W_neigh.astype(jnp.bfloat16))


# speedup vs baseline: 6.1150x; 1.0617x over previous
import functools

import jax
import jax.numpy as jnp
from jax import lax
from jax.experimental import pallas as pl
from jax.experimental.pallas import tpu as pltpu
from jax.experimental.pallas import tpu_sc as plsc

N = 10000
T = 16
D = 256
G = 3 * D
E = N * T

BN = 2000
NB = N // BN

_EPS = 1e-5


def _prep_body(feat_ref, gamma_ref, beta_ref, w_ih_ref, b_ih_ref,
               w_self_ref, feat_bf_ref, feat_pk_ref, w_ihs_ref, bias_ih_ref,
               w_selfs_ref, bias_self_ref):
    f = feat_ref[...]
    mean = jnp.mean(f, axis=0, keepdims=True)
    var = jnp.mean(f * f, axis=0, keepdims=True) - mean * mean
    scale = gamma_ref[...] * lax.rsqrt(var + _EPS)
    shift = beta_ref[...] - mean * scale
    feat_bf_ref[...] = f.astype(jnp.bfloat16)
    u = lax.bitcast_convert_type(f, jnp.int32)
    ubf = jnp.right_shift(u + 0x7FFF + (jnp.right_shift(u, 16) & 1), 16)
    feat_pk_ref[...] = pltpu.pack_elementwise(
        [ubf[:, :D // 2], ubf[:, D // 2:]], packed_dtype=jnp.int16)
    w_ih = w_ih_ref[...]
    w_ihs_ref[...] = (w_ih * scale).astype(jnp.bfloat16)
    bias_ih_ref[...] = b_ih_ref[...] + lax.dot_general(
        shift, w_ih, (((1,), (1,)), ((), ())),
        preferred_element_type=jnp.float32)
    w_self = w_self_ref[...]
    w_selfs_ref[...] = (w_self * scale).astype(jnp.bfloat16)
    bias_self_ref[...] = lax.dot_general(
        shift, w_self, (((1,), (1,)), ((), ())),
        preferred_element_type=jnp.float32)


def _prep(feat, gamma2, beta2, w_ih, b_ih2, w_self):
    return pl.pallas_call(
        _prep_body,
        out_shape=(
            jax.ShapeDtypeStruct((N, D), jnp.bfloat16),
            jax.ShapeDtypeStruct((N, D // 2), jnp.int32),
            jax.ShapeDtypeStruct((G, D), jnp.bfloat16),
            jax.ShapeDtypeStruct((1, G), jnp.float32),
            jax.ShapeDtypeStruct((D, D), jnp.bfloat16),
            jax.ShapeDtypeStruct((1, D), jnp.float32),
        ),
    )(feat, gamma2, beta2, w_ih, b_ih2, w_self)


_NC, _NS = 2, 16
NW = _NC * _NS
TSPLIT = 2
EH = E // TSPLIT
CH = 128
NCHUNK = EH // CH
NK = -(-NCHUNK // NW)


def _gather_body(src_hbm, idx_hbm, out_hbm, idx_v, buf0, buf1,
                 gsem0, gsem1):
    wid = lax.axis_index("s") * _NC + lax.axis_index("c")
    nk_me = jnp.where(wid < NCHUNK - NW * (NK - 1), NK, NK - 1)
    pltpu.sync_copy(idx_hbm.at[wid], idx_v)

    bufs = ((buf0, gsem0), (buf1, gsem1))
    pltpu.async_copy(src_hbm.at[idx_v.at[0]], buf0, gsem0)
    pltpu.async_copy(src_hbm.at[idx_v.at[1]], buf1, gsem1)
    for k in range(NK):
        buf, gsem = bufs[k % 2]
        row = (wid + NW * k) * CH

        @pl.when(k < nk_me)
        def _():
            pltpu.make_async_copy(
                src_hbm.at[idx_v.at[k]], buf, gsem).wait()
            pltpu.sync_copy(buf, out_hbm.at[pl.ds(row, CH)])

        if k + 2 < NK:
            @pl.when(k + 2 < nk_me)
            def _():
                pltpu.async_copy(src_hbm.at[idx_v.at[k + 2]], buf, gsem)


@functools.cache
def _gather():
    return pl.kernel(
        _gather_body,
        mesh=plsc.VectorSubcoreMesh(core_axis_name="c", subcore_axis_name="s",
                                    num_cores=_NC, num_subcores=_NS),
        out_type=jax.ShapeDtypeStruct((EH, D // 2), jnp.int32),
        scratch_types=[
            pltpu.VMEM((NK, CH), jnp.int32),
            pltpu.VMEM((CH, D // 2), jnp.int32),
            pltpu.VMEM((CH, D // 2), jnp.int32),
            pltpu.SemaphoreType.DMA,
            pltpu.SemaphoreType.DMA,
        ],
    )


TH = T // TSPLIT


def _gru_steps(mail_ref, h, w_ihs_ref, bias_ih_ref, whh_ref, bhh_ref):
    for tt in range(TH):
        pk = mail_ref[tt]
        lo = pltpu.unpack_elementwise(
            pk, index=0, packed_dtype=jnp.int16, unpacked_dtype=jnp.int32)
        hi = pltpu.unpack_elementwise(
            pk, index=1, packed_dtype=jnp.int16, unpacked_dtype=jnp.int32)
        mail_bf = jnp.concatenate(
            [lax.bitcast_convert_type(lax.shift_left(lo, 16), jnp.float32),
             lax.bitcast_convert_type(lax.shift_left(hi, 16), jnp.float32)],
            axis=1).astype(jnp.bfloat16)
        x = lax.dot_general(
            mail_bf, w_ihs_ref[...],
            (((1,), (1,)), ((), ())),
            preferred_element_type=jnp.float32) + bias_ih_ref[...]
        gh = lax.dot_general(
            h.astype(jnp.bfloat16), whh_ref[...], (((1,), (1,)), ((), ())),
            preferred_element_type=jnp.float32) + bhh_ref[...]
        rz = x[:, :2 * D] + gh[:, :2 * D]
        srz = 0.5 * jnp.tanh(0.5 * rz) + 0.5
        r = srz[:, :D]
        z = srz[:, D:]
        n = jnp.tanh(x[:, 2 * D:] + r * gh[:, 2 * D:])
        h = z * (h - n) + n
    return h


def _gru1_body(mail_ref, w_ihs_ref, bias_ih_ref, whh_ref, bhh_ref,
               hout_ref):
    h = jnp.zeros((BN, D), jnp.float32)
    hout_ref[...] = _gru_steps(mail_ref, h, w_ihs_ref, bias_ih_ref,
                               whh_ref, bhh_ref)


def _gru2_body(mail_ref, hin_ref, feat_bf_ref, w_ihs_ref, bias_ih_ref,
               whh_ref, bhh_ref, w_selfs_ref, bias_self_ref, wneigh_ref,
               out_ref):
    h_new = _gru_steps(mail_ref, hin_ref[...], w_ihs_ref, bias_ih_ref,
                       whh_ref, bhh_ref)
    out_ref[...] = (
        lax.dot_general(feat_bf_ref[...], w_selfs_ref[...],
                        (((1,), (1,)), ((), ())),
                        preferred_element_type=jnp.float32)
        + bias_self_ref[...]
        + lax.dot_general(h_new.astype(jnp.bfloat16), wneigh_ref[...],
                          (((1,), (1,)), ((), ())),
                          preferred_element_type=jnp.float32))


_SMALL_SPECS = [
    pl.BlockSpec((G, D), lambda i: (0, 0)),
    pl.BlockSpec((1, G), lambda i: (0, 0)),
    pl.BlockSpec((G, D), lambda i: (0, 0)),
    pl.BlockSpec((1, G), lambda i: (0, 0)),
]


def _gru1(mailA, w_ihs, bias_ih, w_hh_bf, b_hh2):
    return pl.pallas_call(
        _gru1_body,
        grid=(NB,),
        in_specs=[pl.BlockSpec((TH, BN, D // 2), lambda i: (0, i, 0))]
        + _SMALL_SPECS,
        out_specs=pl.BlockSpec((BN, D), lambda i: (i, 0)),
        out_shape=jax.ShapeDtypeStruct((N, D), jnp.float32),
        compiler_params=pltpu.CompilerParams(
            dimension_semantics=("arbitrary",)),
    )(mailA, w_ihs, bias_ih, w_hh_bf, b_hh2)


def _gru2(mailB, h_mid, feat_bf, w_ihs, bias_ih, w_hh_bf, b_hh2,
          w_selfs, bias_self, w_neigh_bf):
    return pl.pallas_call(
        _gru2_body,
        grid=(NB,),
        in_specs=[
            pl.BlockSpec((TH, BN, D // 2), lambda i: (0, i, 0)),
            pl.BlockSpec((BN, D), lambda i: (i, 0)),
            pl.BlockSpec((BN, D), lambda i: (i, 0)),
        ] + _SMALL_SPECS + [
            pl.BlockSpec((D, D), lambda i: (0, 0)),
            pl.BlockSpec((1, D), lambda i: (0, 0)),
            pl.BlockSpec((D, D), lambda i: (0, 0)),
        ],
        out_specs=pl.BlockSpec((BN, D), lambda i: (i, 0)),
        out_shape=jax.ShapeDtypeStruct((N, D), jnp.float32),
        compiler_params=pltpu.CompilerParams(
            dimension_semantics=("arbitrary",)),
    )(mailB, h_mid, feat_bf, w_ihs, bias_ih, w_hh_bf, b_hh2,
      w_selfs, bias_self, w_neigh_bf)


def kernel(feat, edge_index, gamma, beta, W_ih, W_hh, b_ih, b_hh,
           W_self, W_neigh):
    src = edge_index[0].astype(jnp.int32)
    src_t = src.reshape(N, T).T.reshape(E)
    cid = jnp.minimum(jnp.arange(NW)[:, None] + NW * jnp.arange(NK)[None, :],
                      NCHUNK - 1)
    src_a = src_t[:EH].reshape(NCHUNK, CH)[cid]
    src_b = src_t[EH:].reshape(NCHUNK, CH)[cid]

    feat_bf, feat_pk, w_ihs, bias_ih, w_selfs, bias_self = _prep(
        feat, gamma.reshape(1, D), beta.reshape(1, D), W_ih,
        b_ih.reshape(1, G), W_self)
    mail_a = _gather()(feat_pk, src_a).reshape(TH, N, D // 2)
    mail_b = _gather()(feat_pk, src_b).reshape(TH, N, D // 2)
    w_hh_bf = W_hh.astype(jnp.bfloat16)
    b_hh2 = b_hh.reshape(1, G)
    h_mid = _gru1(mail_a, w_ihs, bias_ih, w_hh_bf, b_hh2)
    return _gru2(mail_b, h_mid, feat_bf, w_ihs, bias_ih, w_hh_bf, b_hh2,
                 w_selfs, bias_self, W_neigh.astype(jnp.bfloat16))


# speedup vs baseline: 6.5024x; 1.0633x over previous
import functools

import jax
import jax.numpy as jnp
from jax import lax
from jax.experimental import pallas as pl
from jax.experimental.pallas import tpu as pltpu
from jax.experimental.pallas import tpu_sc as plsc

N = 10000
T = 16
D = 256
G = 3 * D
E = N * T

BN = 2000
NB = N // BN

_EPS = 1e-5


def _prep_body(feat_ref, gamma_ref, beta_ref, w_ih_ref, b_ih_ref,
               w_self_ref, feat_bf_ref, feat_pk_ref, w_ihs_ref, bias_ih_ref,
               w_selfs_ref, bias_self_ref):
    f = feat_ref[...]
    mean = jnp.mean(f, axis=0, keepdims=True)
    var = jnp.mean(f * f, axis=0, keepdims=True) - mean * mean
    scale = gamma_ref[...] * lax.rsqrt(var + _EPS)
    shift = beta_ref[...] - mean * scale
    feat_bf_ref[...] = f.astype(jnp.bfloat16)
    u = lax.bitcast_convert_type(f, jnp.int32)
    ubf = jnp.right_shift(u + 0x7FFF + (jnp.right_shift(u, 16) & 1), 16)
    feat_pk_ref[...] = pltpu.pack_elementwise(
        [ubf[:, :D // 2], ubf[:, D // 2:]], packed_dtype=jnp.int16)
    w_ih = w_ih_ref[...]
    w_ihs_ref[...] = (w_ih * scale).astype(jnp.bfloat16)
    bias_ih_ref[...] = b_ih_ref[...] + lax.dot_general(
        shift, w_ih, (((1,), (1,)), ((), ())),
        preferred_element_type=jnp.float32)
    w_self = w_self_ref[...]
    w_selfs_ref[...] = (w_self * scale).astype(jnp.bfloat16)
    bias_self_ref[...] = lax.dot_general(
        shift, w_self, (((1,), (1,)), ((), ())),
        preferred_element_type=jnp.float32)


def _prep(feat, gamma2, beta2, w_ih, b_ih2, w_self):
    return pl.pallas_call(
        _prep_body,
        out_shape=(
            jax.ShapeDtypeStruct((N, D), jnp.bfloat16),
            jax.ShapeDtypeStruct((N, D // 2), jnp.int32),
            jax.ShapeDtypeStruct((G, D), jnp.bfloat16),
            jax.ShapeDtypeStruct((1, G), jnp.float32),
            jax.ShapeDtypeStruct((D, D), jnp.bfloat16),
            jax.ShapeDtypeStruct((1, D), jnp.float32),
        ),
    )(feat, gamma2, beta2, w_ih, b_ih2, w_self)


_NC, _NS = 2, 16
NW = _NC * _NS
TSPLIT = 4
EH = E // TSPLIT
CH = 160
NCHUNK = EH // CH
NK = -(-NCHUNK // NW)


def _gather_body(src_hbm, idx_hbm, out_hbm, idx_v, buf0, buf1,
                 gsem0, gsem1):
    wid = lax.axis_index("s") * _NC + lax.axis_index("c")
    nk_me = jnp.where(wid < NCHUNK - NW * (NK - 1), NK, NK - 1)
    pltpu.sync_copy(idx_hbm.at[wid], idx_v)

    bufs = ((buf0, gsem0), (buf1, gsem1))
    pltpu.async_copy(src_hbm.at[idx_v.at[pl.ds(0, CH)]], buf0, gsem0)
    pltpu.async_copy(src_hbm.at[idx_v.at[pl.ds(CH, CH)]], buf1, gsem1)
    for k in range(NK):
        buf, gsem = bufs[k % 2]
        row = (wid + NW * k) * CH

        @pl.when(k < nk_me)
        def _():
            pltpu.make_async_copy(
                src_hbm.at[idx_v.at[pl.ds(k * CH, CH)]], buf, gsem).wait()
            pltpu.sync_copy(buf, out_hbm.at[pl.ds(row, CH)])

        if k + 2 < NK:
            @pl.when(k + 2 < nk_me)
            def _():
                pltpu.async_copy(
                    src_hbm.at[idx_v.at[pl.ds((k + 2) * CH, CH)]], buf, gsem)


@functools.cache
def _gather():
    return pl.kernel(
        _gather_body,
        mesh=plsc.VectorSubcoreMesh(core_axis_name="c", subcore_axis_name="s",
                                    num_cores=_NC, num_subcores=_NS),
        out_type=jax.ShapeDtypeStruct((EH, D // 2), jnp.int32),
        scratch_types=[
            pltpu.VMEM((NK * CH,), jnp.int32),
            pltpu.VMEM((CH, D // 2), jnp.int32),
            pltpu.VMEM((CH, D // 2), jnp.int32),
            pltpu.SemaphoreType.DMA,
            pltpu.SemaphoreType.DMA,
        ],
    )


TH = T // TSPLIT


def _gru_steps(mail_ref, h, w_ihs_ref, bias_ih_ref, whh_ref, bhh_ref):
    for tt in range(TH):
        pk = mail_ref[tt]
        lo = pltpu.unpack_elementwise(
            pk, index=0, packed_dtype=jnp.int16, unpacked_dtype=jnp.int32)
        hi = pltpu.unpack_elementwise(
            pk, index=1, packed_dtype=jnp.int16, unpacked_dtype=jnp.int32)
        mail_bf = jnp.concatenate(
            [lax.bitcast_convert_type(lax.shift_left(lo, 16), jnp.float32),
             lax.bitcast_convert_type(lax.shift_left(hi, 16), jnp.float32)],
            axis=1).astype(jnp.bfloat16)
        x = lax.dot_general(
            mail_bf, w_ihs_ref[...],
            (((1,), (1,)), ((), ())),
            preferred_element_type=jnp.float32) + bias_ih_ref[...]
        gh = lax.dot_general(
            h.astype(jnp.bfloat16), whh_ref[...], (((1,), (1,)), ((), ())),
            preferred_element_type=jnp.float32) + bhh_ref[...]
        rz = x[:, :2 * D] + gh[:, :2 * D]
        srz = 0.5 * jnp.tanh(0.5 * rz) + 0.5
        r = srz[:, :D]
        z = srz[:, D:]
        n = jnp.tanh(x[:, 2 * D:] + r * gh[:, 2 * D:])
        h = z * (h - n) + n
    return h


def _gru1_body(mail_ref, w_ihs_ref, bias_ih_ref, whh_ref, bhh_ref,
               hout_ref):
    h = jnp.zeros((BN, D), jnp.float32)
    hout_ref[...] = _gru_steps(mail_ref, h, w_ihs_ref, bias_ih_ref,
                               whh_ref, bhh_ref)


def _gru_mid_body(mail_ref, hin_ref, w_ihs_ref, bias_ih_ref, whh_ref,
                  bhh_ref, hout_ref):
    hout_ref[...] = _gru_steps(mail_ref, hin_ref[...], w_ihs_ref,
                               bias_ih_ref, whh_ref, bhh_ref)


def _gru2_body(mail_ref, hin_ref, feat_bf_ref, w_ihs_ref, bias_ih_ref,
               whh_ref, bhh_ref, w_selfs_ref, bias_self_ref, wneigh_ref,
               out_ref):
    h_new = _gru_steps(mail_ref, hin_ref[...], w_ihs_ref, bias_ih_ref,
                       whh_ref, bhh_ref)
    out_ref[...] = (
        lax.dot_general(feat_bf_ref[...], w_selfs_ref[...],
                        (((1,), (1,)), ((), ())),
                        preferred_element_type=jnp.float32)
        + bias_self_ref[...]
        + lax.dot_general(h_new.astype(jnp.bfloat16), wneigh_ref[...],
                          (((1,), (1,)), ((), ())),
                          preferred_element_type=jnp.float32))


_SMALL_SPECS = [
    pl.BlockSpec((G, D), lambda i: (0, 0)),
    pl.BlockSpec((1, G), lambda i: (0, 0)),
    pl.BlockSpec((G, D), lambda i: (0, 0)),
    pl.BlockSpec((1, G), lambda i: (0, 0)),
]


def _gru1(mailA, w_ihs, bias_ih, w_hh_bf, b_hh2):
    return pl.pallas_call(
        _gru1_body,
        grid=(NB,),
        in_specs=[pl.BlockSpec((TH, BN, D // 2), lambda i: (0, i, 0))]
        + _SMALL_SPECS,
        out_specs=pl.BlockSpec((BN, D), lambda i: (i, 0)),
        out_shape=jax.ShapeDtypeStruct((N, D), jnp.float32),
        compiler_params=pltpu.CompilerParams(
            dimension_semantics=("arbitrary",)),
    )(mailA, w_ihs, bias_ih, w_hh_bf, b_hh2)


def _gru_mid(mailX, h_in, w_ihs, bias_ih, w_hh_bf, b_hh2):
    return pl.pallas_call(
        _gru_mid_body,
        grid=(NB,),
        in_specs=[
            pl.BlockSpec((TH, BN, D // 2), lambda i: (0, i, 0)),
            pl.BlockSpec((BN, D), lambda i: (i, 0)),
        ] + _SMALL_SPECS,
        out_specs=pl.BlockSpec((BN, D), lambda i: (i, 0)),
        out_shape=jax.ShapeDtypeStruct((N, D), jnp.float32),
        compiler_params=pltpu.CompilerParams(
            dimension_semantics=("arbitrary",)),
    )(mailX, h_in, w_ihs, bias_ih, w_hh_bf, b_hh2)


def _gru2(mailB, h_mid, feat_bf, w_ihs, bias_ih, w_hh_bf, b_hh2,
          w_selfs, bias_self, w_neigh_bf):
    return pl.pallas_call(
        _gru2_body,
        grid=(NB,),
        in_specs=[
            pl.BlockSpec((TH, BN, D // 2), lambda i: (0, i, 0)),
            pl.BlockSpec((BN, D), lambda i: (i, 0)),
            pl.BlockSpec((BN, D), lambda i: (i, 0)),
        ] + _SMALL_SPECS + [
            pl.BlockSpec((D, D), lambda i: (0, 0)),
            pl.BlockSpec((1, D), lambda i: (0, 0)),
            pl.BlockSpec((D, D), lambda i: (0, 0)),
        ],
        out_specs=pl.BlockSpec((BN, D), lambda i: (i, 0)),
        out_shape=jax.ShapeDtypeStruct((N, D), jnp.float32),
        compiler_params=pltpu.CompilerParams(
            dimension_semantics=("arbitrary",)),
    )(mailB, h_mid, feat_bf, w_ihs, bias_ih, w_hh_bf, b_hh2,
      w_selfs, bias_self, w_neigh_bf)


def kernel(feat, edge_index, gamma, beta, W_ih, W_hh, b_ih, b_hh,
           W_self, W_neigh):
    src = edge_index[0].astype(jnp.int32)
    src_t = src.reshape(N, T).T.reshape(E)
    cid = jnp.minimum(jnp.arange(NW)[:, None] + NW * jnp.arange(NK)[None, :],
                      NCHUNK - 1)
    srcs = [src_t[q * EH:(q + 1) * EH].reshape(NCHUNK, CH)[cid]
            .reshape(NW, NK * CH) for q in range(TSPLIT)]

    feat_bf, feat_pk, w_ihs, bias_ih, w_selfs, bias_self = _prep(
        feat, gamma.reshape(1, D), beta.reshape(1, D), W_ih,
        b_ih.reshape(1, G), W_self)
    mails = [_gather()(feat_pk, s_).reshape(TH, N, D // 2) for s_ in srcs]
    w_hh_bf = W_hh.astype(jnp.bfloat16)
    b_hh2 = b_hh.reshape(1, G)
    h = _gru1(mails[0], w_ihs, bias_ih, w_hh_bf, b_hh2)
    for q in range(1, TSPLIT - 1):
        h = _gru_mid(mails[q], h, w_ihs, bias_ih, w_hh_bf, b_hh2)
    return _gru2(mails[TSPLIT - 1], h, feat_bf, w_ihs, bias_ih, w_hh_bf,
                 b_hh2, w_selfs, bias_self, W_neigh.astype(jnp.bfloat16))
